# Initial kernel scaffold; baseline (speedup 1.0000x reference)
import jax
import jax.numpy as jnp
from jax.experimental import pallas as pl


def kernel(input, target, db_X, db_Y):
    raise NotImplementedError("write your pallas kernel here")


# speedup vs baseline: 1.6740x; 1.6740x over previous
import functools

import jax
import jax.numpy as jnp
from jax.experimental import pallas as pl
from jax.experimental.pallas import tpu as pltpu

K = 100
D = 64
RIDGE_C = 1e-4
N_DB = 100000
NP = 102400
BJ = 2048
QB = 128
KP = 112
GW = 80


def _score_kernel(q_ref, x_ref, s_ref):
    j = pl.program_id(0)
    x = x_ref[...]
    q = q_ref[...]
    g = jax.lax.dot_general(q, x, (((1,), (1,)), ((), ())),
                            precision=jax.lax.Precision.HIGHEST,
                            preferred_element_type=jnp.float32)
    ones = jnp.ones((1, D), jnp.float32)
    n = jax.lax.dot_general(ones, x * x, (((1,), (1,)), ((), ())),
                            precision=jax.lax.Precision.HIGHEST,
                            preferred_element_type=jnp.float32)
    idx = jax.lax.broadcasted_iota(jnp.int32, (1, BJ), 1) + j * BJ
    pad = jnp.where(idx >= N_DB, jnp.float32(1e30), jnp.float32(0.0))
    s_ref[...] = n - 2.0 * g + pad


def _scores(q):
    def call(db_pad):
        return pl.pallas_call(
            _score_kernel,
            grid=(NP // BJ,),
            in_specs=[
                pl.BlockSpec((1024, D), lambda j: (0, 0)),
                pl.BlockSpec((BJ, D), lambda j: (j, 0)),
            ],
            out_specs=pl.BlockSpec((1024, BJ), lambda j: (0, j)),
            out_shape=jax.ShapeDtypeStruct((1024, NP), jnp.float32),
        )(q, db_pad)
    return call


def _reg_kernel(g_ref, q_ref, t_ref, o_ref):
    b = pl.program_id(0)
    g = g_ref[...]
    rowmask = (jax.lax.broadcasted_iota(jnp.int32, (1, KP, 1), 1)
               < K).astype(jnp.float32)
    X = g[:, :, :D] * rowmask
    y = g[:, :, D:D + 1] * rowmask
    ones_col = jnp.broadcast_to(rowmask, (QB, KP, 1))
    Xa = jnp.concatenate([X, ones_col], axis=2)
    Xay = jnp.concatenate([X, ones_col, y], axis=2)
    G = jax.lax.dot_general(Xa, Xay, (((1,), (1,)), ((0,), (0,))),
                            precision=jax.lax.Precision.HIGHEST,
                            preferred_element_type=jnp.float32)
    i1 = jax.lax.broadcasted_iota(jnp.int32, (1, 65, 66), 1)
    i2 = jax.lax.broadcasted_iota(jnp.int32, (1, 65, 66), 2)
    G = G + jnp.where((i1 == i2) & (i2 < 65), jnp.float32(RIDGE_C),
                      jnp.float32(0.0))
    ri = jax.lax.broadcasted_iota(jnp.int32, (1, 65, 1), 1)
    for p in range(65):
        piv = G[:, p:p + 1, :]
        d = G[:, p:p + 1, p:p + 1]
        pivn = piv / d
        col = G[:, :, p:p + 1]
        G = jnp.where(ri == p, pivn, G - col * pivn)
    w = G[:, :, 65:66]
    q = q_ref[...]
    qa = jnp.concatenate([q, jnp.ones((QB, 1), jnp.float32)], axis=1)
    pred = jnp.sum(qa[:, :, None] * w, axis=1)
    t = t_ref[...]
    part = jnp.sum((pred - t) ** 2) / jnp.float32(1024.0)
    prev = jnp.where(b == 0, jnp.float32(0.0), o_ref[0, 0])
    o_ref[...] = (prev + part).reshape(1, 1)


def _regression(gathered, q, t):
    return pl.pallas_call(
        _reg_kernel,
        grid=(1024 // QB,),
        in_specs=[
            pl.BlockSpec((QB, KP, GW), lambda b: (b, 0, 0)),
            pl.BlockSpec((QB, D), lambda b: (b, 0)),
            pl.BlockSpec((QB, 1), lambda b: (b, 0)),
        ],
        out_specs=pl.BlockSpec((1, 1), lambda b: (0, 0)),
        out_shape=jax.ShapeDtypeStruct((1, 1), jnp.float32),
    )(gathered, q, t)


def kernel(input, target, db_X, db_Y):
    db_pad = jnp.pad(db_X, ((0, NP - N_DB), (0, 0)))
    s = _scores(input)(db_pad)
    _, idx = jax.lax.top_k(-s, K)
    xn = jnp.take(db_X, idx, axis=0)
    yn = jnp.take(db_Y[:, 0], idx, axis=0)
    gathered = jnp.concatenate(
        [xn, yn[:, :, None],
         jnp.zeros((1024, K, GW - D - 1), jnp.float32)], axis=2)
    gathered = jnp.pad(gathered, ((0, 0), (0, KP - K), (0, 0)))
    out = _regression(gathered, input, target)
    return out[0, 0]


# speedup vs baseline: 4.3546x; 2.6014x over previous
import functools

import jax
import jax.numpy as jnp
from jax import lax
from jax.experimental import pallas as pl
from jax.experimental.pallas import tpu as pltpu
from jax.experimental.pallas import tpu_sc as plsc

K = 100
D = 64
RIDGE_C = 1e-4
N_DB = 100000
NP = 102400
BJ = 2048
QB = 128
KP = 128
GW = 128
CAP = 4096
NW = 32
QW = 1024 // NW
MININT = -2147483648


def _score_kernel(q_ref, x_ref, s_ref):
    j = pl.program_id(0)
    x = x_ref[...]
    q = q_ref[...]
    g = jax.lax.dot_general(q, x, (((1,), (1,)), ((), ())),
                            precision=jax.lax.Precision.HIGHEST,
                            preferred_element_type=jnp.float32)
    ones = jnp.ones((1, D), jnp.float32)
    n = jax.lax.dot_general(ones, x * x, (((1,), (1,)), ((), ())),
                            precision=jax.lax.Precision.HIGHEST,
                            preferred_element_type=jnp.float32)
    idx = jax.lax.broadcasted_iota(jnp.int32, (1, BJ), 1) + j * BJ
    pad = jnp.where(idx >= N_DB, jnp.float32(1e30), jnp.float32(0.0))
    s_ref[...] = n - 2.0 * g + pad


def _scores(q):
    def call(db_pad):
        return pl.pallas_call(
            _score_kernel,
            grid=(NP // BJ,),
            in_specs=[
                pl.BlockSpec((1024, D), lambda j: (0, 0)),
                pl.BlockSpec((BJ, D), lambda j: (j, 0)),
            ],
            out_specs=pl.BlockSpec((1024, BJ), lambda j: (0, j)),
            out_shape=jax.ShapeDtypeStruct((1024, NP), jnp.float32),
        )(q, db_pad)
    return call


def _reg_kernel(g_ref, q_ref, t_ref, o_ref):
    b = pl.program_id(0)
    g = g_ref[...]
    rowmask = (jax.lax.broadcasted_iota(jnp.int32, (1, KP, 1), 1)
               < K).astype(jnp.float32)
    X = g[:, :, :D] * rowmask
    y = g[:, :, D:D + 1] * rowmask
    ones_col = jnp.broadcast_to(rowmask, (QB, KP, 1))
    Xa = jnp.concatenate([X, ones_col], axis=2)
    Xay = jnp.concatenate([X, ones_col, y], axis=2)
    G = jax.lax.dot_general(Xa, Xay, (((1,), (1,)), ((0,), (0,))),
                            precision=jax.lax.Precision.HIGHEST,
                            preferred_element_type=jnp.float32)
    i1 = jax.lax.broadcasted_iota(jnp.int32, (1, 65, 66), 1)
    i2 = jax.lax.broadcasted_iota(jnp.int32, (1, 65, 66), 2)
    G = G + jnp.where((i1 == i2) & (i2 < 65), jnp.float32(RIDGE_C),
                      jnp.float32(0.0))
    ri = jax.lax.broadcasted_iota(jnp.int32, (1, 65, 1), 1)
    for p in range(65):
        piv = G[:, p:p + 1, :]
        d = G[:, p:p + 1, p:p + 1]
        pivn = piv / d
        col = G[:, :, p:p + 1]
        G = jnp.where(ri == p, pivn, G - col * pivn)
    w = G[:, :, 65:66]
    q = q_ref[...]
    qa = jnp.concatenate([q, jnp.ones((QB, 1), jnp.float32)], axis=1)
    pred = jnp.sum(qa[:, :, None] * w, axis=1)
    t = t_ref[...]
    part = jnp.sum((pred - t) ** 2) / jnp.float32(1024.0)
    prev = jnp.where(b == 0, jnp.float32(0.0), o_ref[0, 0])
    o_ref[...] = (prev + part).reshape(1, 1)


def _regression(gathered, q, t):
    return pl.pallas_call(
        _reg_kernel,
        grid=(1024 // QB,),
        in_specs=[
            pl.BlockSpec((QB, KP, GW), lambda b: (b, 0, 0)),
            pl.BlockSpec((QB, D), lambda b: (b, 0)),
            pl.BlockSpec((QB, 1), lambda b: (b, 0)),
        ],
        out_specs=pl.BlockSpec((1, 1), lambda b: (0, 0)),
        out_shape=jax.ShapeDtypeStruct((1, 1), jnp.float32),
    )(gathered, q, t)


def _f2i(v):
    x = plsc.bitcast(v, jnp.int32)
    return jnp.where(x >= 0, x, jnp.int32(MININT) - x)


def _sc_body(s_hbm, t_hbm, out_hbm, row, sel, grows, hist, sem):
    wid = lax.axis_index("s") * 2 + lax.axis_index("c")
    lane = lax.broadcasted_iota(jnp.int32, (16,), 0)
    onesv = jnp.ones((16,), jnp.int32)

    def per_query(qi, _):
        qq = wid * QW + qi
        pltpu.sync_copy(s_hbm.at[qq], row)

        def hist_zero(i, _):
            hist[pl.ds(i * 16, 16)] = jnp.zeros((16,), jnp.int32)
            return 0

        def scan_level(nbuck, tk):
            def scan(j, c):
                run, found, bstar, nbelow = c
                h = hist[pl.ds(j * 16, 16)]
                s_in = jnp.sum(h)
                csr = plsc.cumsum(h) + run
                c_end = run + s_in
                hit = jnp.logical_and(jnp.logical_not(found), c_end >= tk)
                p = jnp.sum((csr < tk).astype(jnp.int32))
                nb = jnp.sum(jnp.where(lane < p, h, 0))
                bstar = jnp.where(hit, j * 16 + p, bstar)
                nbelow = jnp.where(hit, run + nb, nbelow)
                return (c_end, jnp.logical_or(found, c_end >= tk),
                        bstar, nbelow)
            _, _, bstar, nbelow = lax.fori_loop(
                0, nbuck // 16, scan, (jnp.int32(0), False,
                                       jnp.int32(0), jnp.int32(0)))
            return bstar, nbelow

        tk = jnp.int32(K)
        lax.fori_loop(0, 128, hist_zero, 0)

        def h1(i, _):
            u = _f2i(row[pl.ds(i * 16, 16)])
            row[pl.ds(i * 16, 16)] = plsc.bitcast(u, jnp.float32)
            b = (u >> 21) + 1024
            plsc.addupdate_scatter(hist, [b], onesv)
            return 0
        lax.fori_loop(0, NP // 16, h1, 0)
        b1, nb1 = scan_level(2048, tk)
        p1 = b1 - 1024
        tk = tk - nb1

        lax.fori_loop(0, 128, hist_zero, 0)
        p1v = jnp.full((16,), p1, jnp.int32)

        def h2(i, _):
            u = plsc.bitcast(row[pl.ds(i * 16, 16)], jnp.int32)
            b = (u >> 10) & 0x7FF
            plsc.addupdate_scatter(hist, [b], onesv,
                                   mask=(u >> 21) == p1v)
            return 0
        lax.fori_loop(0, NP // 16, h2, 0)
        b2, nb2 = scan_level(2048, tk)
        p2 = (p1 << 11) | b2
        tk = tk - nb2

        lax.fori_loop(0, 64, hist_zero, 0)
        p2v = jnp.full((16,), p2, jnp.int32)

        def h3(i, _):
            u = plsc.bitcast(row[pl.ds(i * 16, 16)], jnp.int32)
            b = u & 0x3FF
            plsc.addupdate_scatter(hist, [b], onesv,
                                   mask=(u >> 10) == p2v)
            return 0
        lax.fori_loop(0, NP // 16, h3, 0)
        b3, nb3 = scan_level(1024, tk)
        ustar = (p2 << 10) | b3
        tq = tk - nb3
        usv = jnp.full((16,), ustar, jnp.int32)
        tqv = jnp.full((16,), tq, jnp.int32)

        def zsel(i, _):
            sel[pl.ds(i * 16, 16)] = jnp.zeros((16,), jnp.int32)
            return 0
        lax.fori_loop(0, KP // 16, zsel, 0)

        def emit(i, c):
            off2, eqrun = c
            u = plsc.bitcast(row[pl.ds(i * 16, 16)], jnp.int32)
            lt = u < usv
            eq = u == usv
            ce = plsc.cumsum(eq.astype(jnp.int32)) + eqrun
            tksel = jnp.logical_or(lt, jnp.logical_and(eq, ce <= tqv))
            offc = jnp.minimum(off2, KP - 16)
            plsc.store_compressed(sel.at[pl.ds(offc, 16)], lane + i * 16,
                                  mask=tksel)
            return (off2 + jnp.sum(tksel.astype(jnp.int32)),
                    eqrun + jnp.sum(eq.astype(jnp.int32)))
        lax.fori_loop(0, NP // 16, emit, (jnp.int32(0), jnp.int32(0)))

        pltpu.async_copy(t_hbm.at[sel], grows, sem).wait()
        pltpu.sync_copy(grows, out_hbm.at[qq])
        return 0

    lax.fori_loop(0, QW, per_query, 0)


def _sc_select_gather(s, table):
    mesh = plsc.VectorSubcoreMesh(core_axis_name="c", subcore_axis_name="s")
    f = pl.kernel(
        _sc_body,
        out_type=jax.ShapeDtypeStruct((1024, KP, GW), jnp.float32),
        mesh=mesh,
        compiler_params=pltpu.CompilerParams(needs_layout_passes=False),
        scratch_types=[
            pltpu.VMEM((NP,), jnp.float32),
            pltpu.VMEM((KP,), jnp.int32),
            pltpu.VMEM((KP, GW), jnp.float32),
            pltpu.VMEM((2048,), jnp.int32),
            pltpu.SemaphoreType.DMA,
        ],
    )
    return f(s, table)


def kernel(input, target, db_X, db_Y):
    db_pad = jnp.pad(db_X, ((0, NP - N_DB), (0, 0)))
    s = _scores(input)(db_pad)
    table = jnp.concatenate(
        [db_X, db_Y, jnp.zeros((N_DB, GW - D - 1), jnp.float32)], axis=1)
    gathered = _sc_select_gather(s, table)
    out = _regression(gathered, input, target)
    return out[0, 0]


# speedup vs baseline: 6.2627x; 1.4382x over previous
import functools

import jax
import jax.numpy as jnp
from jax import lax
from jax.experimental import pallas as pl
from jax.experimental.pallas import tpu as pltpu
from jax.experimental.pallas import tpu_sc as plsc

K = 100
D = 64
RIDGE_C = 1e-4
N_DB = 100000
NP = 102400
BJ = 2048
QB = 128
KP = 128
GW = 128
CAP = 4096
NW = 32
QW = 1024 // NW
MININT = -2147483648


def _score_kernel(q_ref, x_ref, s_ref):
    j = pl.program_id(0)
    x = x_ref[...]
    q = q_ref[...]
    g = jax.lax.dot_general(q, x, (((1,), (1,)), ((), ())),
                            precision=jax.lax.Precision.HIGHEST,
                            preferred_element_type=jnp.float32)
    ones = jnp.ones((1, D), jnp.float32)
    n = jax.lax.dot_general(ones, x * x, (((1,), (1,)), ((), ())),
                            precision=jax.lax.Precision.HIGHEST,
                            preferred_element_type=jnp.float32)
    idx = jax.lax.broadcasted_iota(jnp.int32, (1, BJ), 1) + j * BJ
    pad = jnp.where(idx >= N_DB, jnp.float32(1e30), jnp.float32(0.0))
    s_ref[...] = n - 2.0 * g + pad


def _scores(q):
    def call(db_pad):
        return pl.pallas_call(
            _score_kernel,
            grid=(NP // BJ,),
            in_specs=[
                pl.BlockSpec((1024, D), lambda j: (0, 0)),
                pl.BlockSpec((BJ, D), lambda j: (j, 0)),
            ],
            out_specs=pl.BlockSpec((1024, BJ), lambda j: (0, j)),
            out_shape=jax.ShapeDtypeStruct((1024, NP), jnp.float32),
        )(q, db_pad)
    return call


def _reg_kernel(g_ref, q_ref, t_ref, o_ref):
    b = pl.program_id(0)
    g = g_ref[...]
    rowmask = (jax.lax.broadcasted_iota(jnp.int32, (1, KP, 1), 1)
               < K).astype(jnp.float32)
    X = g[:, :, :D] * rowmask
    y = g[:, :, D:D + 1] * rowmask
    ones_col = jnp.broadcast_to(rowmask, (QB, KP, 1))
    Xa = jnp.concatenate([X, ones_col], axis=2)
    Xay = jnp.concatenate([X, ones_col, y], axis=2)
    G = jax.lax.dot_general(Xa, Xay, (((1,), (1,)), ((0,), (0,))),
                            precision=jax.lax.Precision.HIGHEST,
                            preferred_element_type=jnp.float32)
    i1 = jax.lax.broadcasted_iota(jnp.int32, (1, 65, 66), 1)
    i2 = jax.lax.broadcasted_iota(jnp.int32, (1, 65, 66), 2)
    G = G + jnp.where((i1 == i2) & (i2 < 65), jnp.float32(RIDGE_C),
                      jnp.float32(0.0))
    ri = jax.lax.broadcasted_iota(jnp.int32, (1, 65, 1), 1)
    for p in range(65):
        piv = G[:, p:p + 1, :]
        d = G[:, p:p + 1, p:p + 1]
        pivn = piv / d
        col = G[:, :, p:p + 1]
        G = jnp.where(ri == p, pivn, G - col * pivn)
    w = G[:, :, 65:66]
    q = q_ref[...]
    qa = jnp.concatenate([q, jnp.ones((QB, 1), jnp.float32)], axis=1)
    pred = jnp.sum(qa[:, :, None] * w, axis=1)
    t = t_ref[...]
    part = jnp.sum((pred - t) ** 2) / jnp.float32(1024.0)
    prev = jnp.where(b == 0, jnp.float32(0.0), o_ref[0, 0])
    o_ref[...] = (prev + part).reshape(1, 1)


def _regression(gathered, q, t):
    return pl.pallas_call(
        _reg_kernel,
        grid=(1024 // QB,),
        in_specs=[
            pl.BlockSpec((QB, KP, GW), lambda b: (b, 0, 0)),
            pl.BlockSpec((QB, D), lambda b: (b, 0)),
            pl.BlockSpec((QB, 1), lambda b: (b, 0)),
        ],
        out_specs=pl.BlockSpec((1, 1), lambda b: (0, 0)),
        out_shape=jax.ShapeDtypeStruct((1, 1), jnp.float32),
    )(gathered, q, t)


def _f2i(v):
    x = plsc.bitcast(v, jnp.int32)
    return jnp.where(x >= 0, x, jnp.int32(MININT) - x)


def _sc_body(s_hbm, t_hbm, out_hbm, row, cv, ci, sel, grows, hist, sem):
    wid = lax.axis_index("s") * 2 + lax.axis_index("c")
    lane = lax.broadcasted_iota(jnp.int32, (16,), 0)
    onesv = jnp.ones((16,), jnp.int32)
    MAXI = 2147483647

    def per_query(qi, _):
        qq = wid * QW + qi
        pltpu.sync_copy(s_hbm.at[qq], row)

        def hist_zero(i, _):
            hist[pl.ds(i * 16, 16)] = jnp.zeros((16,), jnp.int32)
            return 0

        def scan_level(nbuck, tk):
            def scan(j, c):
                run, found, bstar, nbelow = c
                h = hist[pl.ds(j * 16, 16)]
                s_in = jnp.sum(h)
                csr = plsc.cumsum(h) + run
                c_end = run + s_in
                hit = jnp.logical_and(jnp.logical_not(found), c_end >= tk)
                p = jnp.sum((csr < tk).astype(jnp.int32))
                nb = jnp.sum(jnp.where(lane < p, h, 0))
                bstar = jnp.where(hit, j * 16 + p, bstar)
                nbelow = jnp.where(hit, run + nb, nbelow)
                return (c_end, jnp.logical_or(found, c_end >= tk),
                        bstar, nbelow)
            _, _, bstar, nbelow = lax.fori_loop(
                0, nbuck // 16, scan, (jnp.int32(0), False,
                                       jnp.int32(0), jnp.int32(0)))
            return bstar, nbelow

        lax.fori_loop(0, 128, hist_zero, 0)

        def h1(i, _):
            u = _f2i(row[pl.ds(i * 16, 16)])
            row[pl.ds(i * 16, 16)] = plsc.bitcast(u, jnp.float32)
            b = (u >> 21) + 1024
            plsc.addupdate_scatter(hist, [b], onesv)
            return 0
        lax.fori_loop(0, NP // 16, h1, 0)
        b1, nb1 = scan_level(2048, jnp.int32(K))
        tk2 = jnp.int32(K) - nb1
        b1v = jnp.full((16,), b1, jnp.int32)

        def zsel(i, _):
            sel[pl.ds(i * 16, 16)] = jnp.zeros((16,), jnp.int32)
            return 0
        lax.fori_loop(0, KP // 16, zsel, 0)

        def collect(i, c):
            offs, offc = c
            u = plsc.bitcast(row[pl.ds(i * 16, 16)], jnp.int32)
            b = (u >> 21) + 1024
            islt = b < b1v
            isc = b == b1v
            iv = lane + i * 16
            plsc.store_compressed(sel.at[pl.ds(jnp.minimum(offs, KP - 16),
                                               16)], iv, mask=islt)
            okc = jnp.logical_and(isc, offc <= CAP - 16)
            occ = jnp.minimum(offc, CAP - 16)
            plsc.store_compressed(cv.at[pl.ds(occ, 16)], u, mask=okc)
            plsc.store_compressed(ci.at[pl.ds(occ, 16)], iv, mask=okc)
            return (offs + jnp.sum(islt.astype(jnp.int32)),
                    offc + jnp.sum(isc.astype(jnp.int32)))
        offs, nc = lax.fori_loop(0, NP // 16, collect,
                                 (jnp.int32(0), jnp.int32(0)))
        ncc = jnp.minimum(nc, CAP)
        cv[pl.ds(jnp.minimum(ncc, CAP - 16), 16)] = jnp.full(
            (16,), MAXI, jnp.int32)
        trip = (ncc + 15) >> 4

        lax.fori_loop(0, 128, hist_zero, 0)
        maxiv = jnp.full((16,), MAXI, jnp.int32)

        def h2(i, _):
            u = cv[pl.ds(i * 16, 16)]
            b = (u >> 10) & 0x7FF
            plsc.addupdate_scatter(hist, [b], onesv, mask=u != maxiv)
            return 0
        lax.fori_loop(0, trip, h2, 0)
        b2, nb2 = scan_level(2048, tk2)
        p2 = ((b1 - 1024) << 11) | b2
        tk3 = tk2 - nb2

        lax.fori_loop(0, 64, hist_zero, 0)
        p2v = jnp.full((16,), p2, jnp.int32)

        def h3(i, _):
            u = cv[pl.ds(i * 16, 16)]
            b = u & 0x3FF
            plsc.addupdate_scatter(
                hist, [b], onesv,
                mask=jnp.logical_and((u >> 10) == p2v, u != maxiv))
            return 0
        lax.fori_loop(0, trip, h3, 0)
        b3, nb3 = scan_level(1024, tk3)
        ustar = (p2 << 10) | b3
        tq = tk3 - nb3
        usv = jnp.full((16,), ustar, jnp.int32)
        tqv = jnp.full((16,), tq, jnp.int32)

        def emit(i, c):
            off2, eqrun = c
            u = cv[pl.ds(i * 16, 16)]
            iv = ci[pl.ds(i * 16, 16)]
            lt = u < usv
            eq = u == usv
            ce = plsc.cumsum(eq.astype(jnp.int32)) + eqrun
            tksel = jnp.logical_or(lt, jnp.logical_and(eq, ce <= tqv))
            plsc.store_compressed(sel.at[pl.ds(jnp.minimum(off2, KP - 16),
                                               16)], iv, mask=tksel)
            return (off2 + jnp.sum(tksel.astype(jnp.int32)),
                    eqrun + jnp.sum(eq.astype(jnp.int32)))
        lax.fori_loop(0, trip, emit, (offs, jnp.int32(0)))

        pltpu.async_copy(t_hbm.at[sel], grows, sem).wait()
        pltpu.sync_copy(grows, out_hbm.at[qq])
        return 0

    lax.fori_loop(0, QW, per_query, 0)


def _sc_select_gather(s, table):
    mesh = plsc.VectorSubcoreMesh(core_axis_name="c", subcore_axis_name="s")
    f = pl.kernel(
        _sc_body,
        out_type=jax.ShapeDtypeStruct((1024, KP, GW), jnp.float32),
        mesh=mesh,
        compiler_params=pltpu.CompilerParams(needs_layout_passes=False),
        scratch_types=[
            pltpu.VMEM((NP,), jnp.float32),
            pltpu.VMEM((CAP,), jnp.int32),
            pltpu.VMEM((CAP,), jnp.int32),
            pltpu.VMEM((KP,), jnp.int32),
            pltpu.VMEM((KP, GW), jnp.float32),
            pltpu.VMEM((2048,), jnp.int32),
            pltpu.SemaphoreType.DMA,
        ],
    )
    return f(s, table)


def kernel(input, target, db_X, db_Y):
    db_pad = jnp.pad(db_X, ((0, NP - N_DB), (0, 0)))
    s = _scores(input)(db_pad)
    table = jnp.concatenate(
        [db_X, db_Y, jnp.zeros((N_DB, GW - D - 1), jnp.float32)], axis=1)
    gathered = _sc_select_gather(s, table)
    out = _regression(gathered, input, target)
    return out[0, 0]


# speedup vs baseline: 12.5008x; 1.9961x over previous
import functools

import jax
import jax.numpy as jnp
from jax import lax
from jax.experimental import pallas as pl
from jax.experimental.pallas import tpu as pltpu
from jax.experimental.pallas import tpu_sc as plsc

K = 100
D = 64
RIDGE_C = 1e-4
N_DB = 100000
NP = 102400
BJ = 2048
QB = 128
KP = 128
GW = 128
CAP = 4096
NW = 32
QW = 1024 // NW
MININT = -2147483648


def _score_kernel(q_ref, x_ref, s_ref):
    j = pl.program_id(0)
    x = x_ref[...]
    q = q_ref[...]
    g = jax.lax.dot_general(q, x, (((1,), (1,)), ((), ())),
                            precision=jax.lax.Precision.HIGHEST,
                            preferred_element_type=jnp.float32)
    ones = jnp.ones((1, D), jnp.float32)
    n = jax.lax.dot_general(ones, x * x, (((1,), (1,)), ((), ())),
                            precision=jax.lax.Precision.HIGHEST,
                            preferred_element_type=jnp.float32)
    idx = jax.lax.broadcasted_iota(jnp.int32, (1, BJ), 1) + j * BJ
    pad = jnp.where(idx >= N_DB, jnp.float32(1e30), jnp.float32(0.0))
    s_ref[...] = n - 2.0 * g + pad


def _scores(q):
    def call(db_pad):
        return pl.pallas_call(
            _score_kernel,
            grid=(NP // BJ,),
            in_specs=[
                pl.BlockSpec((1024, D), lambda j: (0, 0)),
                pl.BlockSpec((BJ, D), lambda j: (j, 0)),
            ],
            out_specs=pl.BlockSpec((1024, BJ), lambda j: (0, j)),
            out_shape=jax.ShapeDtypeStruct((1024, NP), jnp.float32),
        )(q, db_pad)
    return call


def _reg_kernel(g_ref, q_ref, t_ref, o_ref):
    b = pl.program_id(0)
    g = g_ref[...]
    rowmask = (jax.lax.broadcasted_iota(jnp.int32, (1, KP, 1), 1)
               < K).astype(jnp.float32)
    X = g[:, :, :D] * rowmask
    y = g[:, :, D:D + 1] * rowmask
    ones_col = jnp.broadcast_to(rowmask, (QB, KP, 1))
    Xa = jnp.concatenate([X, ones_col], axis=2)
    Xay = jnp.concatenate([X, ones_col, y], axis=2)
    G = jax.lax.dot_general(Xa, Xay, (((1,), (1,)), ((0,), (0,))),
                            precision=jax.lax.Precision.HIGHEST,
                            preferred_element_type=jnp.float32)
    i1 = jax.lax.broadcasted_iota(jnp.int32, (1, 65, 66), 1)
    i2 = jax.lax.broadcasted_iota(jnp.int32, (1, 65, 66), 2)
    G = G + jnp.where((i1 == i2) & (i2 < 65), jnp.float32(RIDGE_C),
                      jnp.float32(0.0))
    ri = jax.lax.broadcasted_iota(jnp.int32, (1, 65, 1), 1)
    for p in range(65):
        piv = G[:, p:p + 1, :]
        d = G[:, p:p + 1, p:p + 1]
        pivn = piv / d
        col = G[:, :, p:p + 1]
        G = jnp.where(ri == p, pivn, G - col * pivn)
    w = G[:, :, 65:66]
    q = q_ref[...]
    qa = jnp.concatenate([q, jnp.ones((QB, 1), jnp.float32)], axis=1)
    pred = jnp.sum(qa[:, :, None] * w, axis=1)
    t = t_ref[...]
    part = jnp.sum((pred - t) ** 2) / jnp.float32(1024.0)
    prev = jnp.where(b == 0, jnp.float32(0.0), o_ref[0, 0])
    o_ref[...] = (prev + part).reshape(1, 1)


def _regression(gathered, q, t):
    return pl.pallas_call(
        _reg_kernel,
        grid=(1024 // QB,),
        in_specs=[
            pl.BlockSpec((QB, KP, GW), lambda b: (b, 0, 0)),
            pl.BlockSpec((QB, D), lambda b: (b, 0)),
            pl.BlockSpec((QB, 1), lambda b: (b, 0)),
        ],
        out_specs=pl.BlockSpec((1, 1), lambda b: (0, 0)),
        out_shape=jax.ShapeDtypeStruct((1, 1), jnp.float32),
    )(gathered, q, t)


def _f2i(v):
    x = plsc.bitcast(v, jnp.int32)
    return jnp.where(x >= 0, x, jnp.int32(MININT) - x)


def _sc_body(s_hbm, t_hbm, out_hbm, row, cv, ci, sel, grows, hist, sem):
    wid = lax.axis_index("s") * 2 + lax.axis_index("c")
    lane = lax.broadcasted_iota(jnp.int32, (16,), 0)
    onesv = jnp.ones((16,), jnp.int32)
    MAXI = 2147483647

    def per_query(qi, _):
        qq = wid * QW + qi
        pltpu.sync_copy(s_hbm.at[qq], row)

        def hist_zero(i, _):
            hist[pl.ds(i * 16, 16)] = jnp.zeros((16,), jnp.int32)
            return 0

        def scan_level(nbuck, tk):
            def scan(j, c):
                run, found, bstar, nbelow = c
                h = hist[pl.ds(j * 16, 16)]
                s_in = jnp.sum(h)
                csr = plsc.cumsum(h) + run
                c_end = run + s_in
                hit = jnp.logical_and(jnp.logical_not(found), c_end >= tk)
                p = jnp.sum((csr < tk).astype(jnp.int32))
                nb = jnp.sum(jnp.where(lane < p, h, 0))
                bstar = jnp.where(hit, j * 16 + p, bstar)
                nbelow = jnp.where(hit, run + nb, nbelow)
                return (c_end, jnp.logical_or(found, c_end >= tk),
                        bstar, nbelow)
            _, _, bstar, nbelow = lax.fori_loop(
                0, nbuck // 16, scan, (jnp.int32(0), False,
                                       jnp.int32(0), jnp.int32(0)))
            return bstar, nbelow

        lax.fori_loop(0, 128, hist_zero, 0)

        @plsc.parallel_loop(0, NP // 16, unroll=8)
        def h1(i):
            u = _f2i(row[pl.ds(i * 16, 16)])
            row[pl.ds(i * 16, 16)] = plsc.bitcast(u, jnp.float32)
            b = (u >> 21) + 1024
            plsc.addupdate_scatter(hist, [b], onesv)
        b1, nb1 = scan_level(2048, jnp.int32(K))
        tk2 = jnp.int32(K) - nb1
        b1v = jnp.full((16,), b1, jnp.int32)

        def zsel(i, _):
            sel[pl.ds(i * 16, 16)] = jnp.zeros((16,), jnp.int32)
            return 0
        lax.fori_loop(0, KP // 16, zsel, 0)

        @plsc.parallel_loop(0, NP // 16, unroll=2,
                            carry=(jnp.int32(0), jnp.int32(0)))
        def collect(i, c):
            offs, offc = c
            u = plsc.bitcast(row[pl.ds(i * 16, 16)], jnp.int32)
            b = (u >> 21) + 1024
            islt = b < b1v
            isc = b == b1v
            iv = lane + i * 16
            plsc.store_compressed(sel.at[pl.ds(jnp.minimum(offs, KP - 16),
                                               16)], iv, mask=islt)
            okc = jnp.logical_and(isc, offc <= CAP - 16)
            occ = jnp.minimum(offc, CAP - 16)
            plsc.store_compressed(cv.at[pl.ds(occ, 16)], u, mask=okc)
            plsc.store_compressed(ci.at[pl.ds(occ, 16)], iv, mask=okc)
            return (offs + jnp.sum(islt.astype(jnp.int32)),
                    offc + jnp.sum(isc.astype(jnp.int32)))
        offs, nc = collect
        ncc = jnp.minimum(nc, CAP)
        cv[pl.ds(jnp.minimum(ncc, CAP - 16), 16)] = jnp.full(
            (16,), MAXI, jnp.int32)
        trip = (ncc + 15) >> 4

        lax.fori_loop(0, 128, hist_zero, 0)
        maxiv = jnp.full((16,), MAXI, jnp.int32)

        def h2(i, _):
            u = cv[pl.ds(i * 16, 16)]
            b = (u >> 10) & 0x7FF
            plsc.addupdate_scatter(hist, [b], onesv, mask=u != maxiv)
            return 0
        lax.fori_loop(0, trip, h2, 0)
        b2, nb2 = scan_level(2048, tk2)
        p2 = ((b1 - 1024) << 11) | b2
        tk3 = tk2 - nb2

        lax.fori_loop(0, 64, hist_zero, 0)
        p2v = jnp.full((16,), p2, jnp.int32)

        def h3(i, _):
            u = cv[pl.ds(i * 16, 16)]
            b = u & 0x3FF
            plsc.addupdate_scatter(
                hist, [b], onesv,
                mask=jnp.logical_and((u >> 10) == p2v, u != maxiv))
            return 0
        lax.fori_loop(0, trip, h3, 0)
        b3, nb3 = scan_level(1024, tk3)
        ustar = (p2 << 10) | b3
        tq = tk3 - nb3
        usv = jnp.full((16,), ustar, jnp.int32)
        tqv = jnp.full((16,), tq, jnp.int32)

        def emit(i, c):
            off2, eqrun = c
            u = cv[pl.ds(i * 16, 16)]
            iv = ci[pl.ds(i * 16, 16)]
            lt = u < usv
            eq = u == usv
            ce = plsc.cumsum(eq.astype(jnp.int32)) + eqrun
            tksel = jnp.logical_or(lt, jnp.logical_and(eq, ce <= tqv))
            plsc.store_compressed(sel.at[pl.ds(jnp.minimum(off2, KP - 16),
                                               16)], iv, mask=tksel)
            return (off2 + jnp.sum(tksel.astype(jnp.int32)),
                    eqrun + jnp.sum(eq.astype(jnp.int32)))
        lax.fori_loop(0, trip, emit, (offs, jnp.int32(0)))

        pltpu.async_copy(t_hbm.at[sel], grows, sem).wait()
        pltpu.sync_copy(grows, out_hbm.at[qq])
        return 0

    lax.fori_loop(0, QW, per_query, 0)


def _sc_select_gather(s, table):
    mesh = plsc.VectorSubcoreMesh(core_axis_name="c", subcore_axis_name="s")
    f = pl.kernel(
        _sc_body,
        out_type=jax.ShapeDtypeStruct((1024, KP, GW), jnp.float32),
        mesh=mesh,
        compiler_params=pltpu.CompilerParams(needs_layout_passes=False),
        scratch_types=[
            pltpu.VMEM((NP,), jnp.float32),
            pltpu.VMEM((CAP,), jnp.int32),
            pltpu.VMEM((CAP,), jnp.int32),
            pltpu.VMEM((KP,), jnp.int32),
            pltpu.VMEM((KP, GW), jnp.float32),
            pltpu.VMEM((2048,), jnp.int32),
            pltpu.SemaphoreType.DMA,
        ],
    )
    return f(s, table)


def kernel(input, target, db_X, db_Y):
    db_pad = jnp.pad(db_X, ((0, NP - N_DB), (0, 0)))
    s = _scores(input)(db_pad)
    table = jnp.concatenate(
        [db_X, db_Y, jnp.zeros((N_DB, GW - D - 1), jnp.float32)], axis=1)
    gathered = _sc_select_gather(s, table)
    out = _regression(gathered, input, target)
    return out[0, 0]
